# baseline (speedup 1.0000x reference)
import functools

import jax
import jax.numpy as jnp
from jax.experimental import pallas as pl
from jax.experimental.pallas import tpu as pltpu

_EPS = 0.1
_IGNORE = 0
_NEG = -3.0e38


def _tct_body(x_ref, lab_ref, num_ref, den_ref, m_s, s_s, t_s, g_s, *, ck, nb):
    i = pl.program_id(0)
    x = x_ref[...]
    lab = lab_ref[0]
    r = x.shape[1]

    @pl.when(i == 0)
    def _init():
        m_s[...] = jnp.full_like(m_s, _NEG)
        s_s[...] = jnp.zeros_like(s_s)
        t_s[...] = jnp.zeros_like(t_s)
        g_s[...] = jnp.zeros_like(g_s)

    m_old = m_s[...]
    mc = jnp.max(x, axis=0, keepdims=True)
    m_new = jnp.maximum(m_old, mc)
    corr = jnp.exp(m_old - m_new)
    s_new = s_s[...] * corr + jnp.sum(jnp.exp(x - m_new), axis=0, keepdims=True)
    m_s[...] = m_new
    s_s[...] = s_new
    t_s[...] += jnp.sum(x, axis=0, keepdims=True)

    rows = jax.lax.broadcasted_iota(jnp.int32, x.shape, 0) + i * ck
    hit = rows == lab
    g_new = g_s[...] + jnp.sum(jnp.where(hit, x, 0.0), axis=0, keepdims=True)
    g_s[...] = g_new

    @pl.when(i == nb - 1)
    def _fin():
        k = nb * ck
        lse = m_new + jnp.log(s_new)
        per = lse - (1.0 - _EPS) * g_new - (_EPS / k) * t_s[...]
        mask = (lab != _IGNORE).astype(jnp.float32)
        num_ref[...] = jnp.sum(per * mask).reshape(1, 1)
        den_ref[...] = jnp.sum(mask).reshape(1, 1)


@functools.partial(jax.jit, static_argnames=("ck",))
def _ce_loss(preds, labels, ck=1000):
    r, k = preds.shape
    pt = preds.T
    nb = k // ck
    lab2 = labels.astype(jnp.int32).reshape(1, r)
    num, den = pl.pallas_call(
        functools.partial(_tct_body, ck=ck, nb=nb),
        grid=(nb,),
        in_specs=[
            pl.BlockSpec((ck, r), lambda i: (i, 0)),
            pl.BlockSpec((1, r), lambda i: (0, 0)),
        ],
        out_specs=[
            pl.BlockSpec((1, 1), lambda i: (0, 0)),
            pl.BlockSpec((1, 1), lambda i: (0, 0)),
        ],
        out_shape=[
            jax.ShapeDtypeStruct((1, 1), jnp.float32),
            jax.ShapeDtypeStruct((1, 1), jnp.float32),
        ],
        scratch_shapes=[
            pltpu.VMEM((1, r), jnp.float32),
            pltpu.VMEM((1, r), jnp.float32),
            pltpu.VMEM((1, r), jnp.float32),
            pltpu.VMEM((1, r), jnp.float32),
        ],
    )(pt, lab2)
    return num[0, 0] / den[0, 0]


def kernel(preds, labels):
    return _ce_loss(preds, labels)
